# baseline (speedup 1.0000x reference)
import functools

import jax
import jax.numpy as jnp
from jax import lax
from jax.experimental import pallas as pl
from jax.experimental.pallas import tpu as pltpu
from jax.experimental.pallas import tpu_sc as plsc

N = 10000
E = 320000
D = 128
NC = 2
NS = 16
L = 16
NW = NC * NS
B = 80
NB = 128
EPW = NB * B
EPAD = NW * EPW - E
NPAD = 10240
ZCH = NPAD // NS
RB = 1000
GRID = N // RB

_mesh = plsc.VectorSubcoreMesh(
    core_axis_name="c", subcore_axis_name="s", num_cores=NC, num_subcores=NS)


def _zero16():
    return jnp.zeros((L,), jnp.float32)


def _deg_body(dst_hbm, ew_hbm, degp_hbm, dst_v, ew_v, zbuf, deg_sh):
    cid = lax.axis_index("c")
    sid = lax.axis_index("s")
    wid = sid * NC + cid
    pltpu.sync_copy(dst_hbm.at[wid], dst_v)
    pltpu.sync_copy(ew_hbm.at[pl.ds(wid * EPW, EPW)], ew_v)

    def zb(t, carry):
        zbuf[pl.ds(t * L, L)] = _zero16()
        return carry
    lax.fori_loop(0, ZCH // L, zb, 0)
    pltpu.sync_copy(zbuf, deg_sh.at[pl.ds(sid * ZCH, ZCH)])
    plsc.subcore_barrier()

    def body(j, carry):
        pltpu.sync_copy(ew_v.at[pl.ds(j * B, B)], deg_sh.at[dst_v.at[j]],
                        add=True)
        return carry
    lax.fori_loop(0, NB, body, 0)
    plsc.subcore_barrier()
    pltpu.sync_copy(deg_sh.at[pl.ds(sid * ZCH, ZCH)],
                    degp_hbm.at[cid, pl.ds(sid * ZCH, ZCH)])


_deg_call = pl.kernel(
    _deg_body,
    out_type=jax.ShapeDtypeStruct((NC, NPAD), jnp.float32),
    mesh=_mesh,
    scratch_types=[
        pltpu.VMEM((NB, B), jnp.int32),
        pltpu.VMEM((EPW,), jnp.float32),
        pltpu.VMEM((ZCH,), jnp.float32),
        pltpu.VMEM_SHARED((NPAD,), jnp.float32),
    ],
)


def _agg_body(src_hbm, dst_hbm, ew_hbm, y_hbm, accp_hbm,
              src_v, dst_v, ew_v, rows, acc_sh, gsem):
    cid = lax.axis_index("c")
    sid = lax.axis_index("s")
    wid = sid * NC + cid
    pltpu.sync_copy(src_hbm.at[pl.ds(wid * EPW, EPW)], src_v)
    pltpu.sync_copy(dst_hbm.at[wid], dst_v)
    pltpu.sync_copy(ew_hbm.at[pl.ds(wid * EPW, EPW)], ew_v)

    def zb(t, carry):
        for k in range(D // L):
            rows[t, pl.ds(k * L, L)] = _zero16()
        return carry
    lax.fori_loop(0, B, zb, 0)
    for z in range(ZCH // B):
        pltpu.sync_copy(rows, acc_sh.at[pl.ds(sid * ZCH + z * B, B)])
    plsc.subcore_barrier()

    def body(j, carry):
        pltpu.async_copy(y_hbm.at[src_v.at[pl.ds(j * B, B)]], rows,
                         gsem).wait()

        def scale(g, c2):
            wvec = ew_v[pl.ds(j * B + g * L, L)]
            for i in range(L):
                wv = lax.broadcast_in_dim(wvec[i], (L,), ())
                e = g * L + i
                for k in range(D // L):
                    sl = pl.ds(k * L, L)
                    rows[e, sl] = rows[e, sl] * wv
            return c2
        lax.fori_loop(0, B // L, scale, 0)
        pltpu.sync_copy(rows, acc_sh.at[dst_v.at[j]], add=True)
        return carry
    lax.fori_loop(0, NB, body, 0)
    plsc.subcore_barrier()
    pltpu.sync_copy(acc_sh.at[pl.ds(sid * ZCH, ZCH)],
                    accp_hbm.at[cid, pl.ds(sid * ZCH, ZCH)])


_agg_call = pl.kernel(
    _agg_body,
    out_type=jax.ShapeDtypeStruct((NC, NPAD, D), jnp.float32),
    mesh=_mesh,
    scratch_types=[
        pltpu.VMEM((EPW,), jnp.int32),
        pltpu.VMEM((NB, B), jnp.int32),
        pltpu.VMEM((EPW,), jnp.float32),
        pltpu.VMEM((B, D), jnp.float32),
        pltpu.VMEM_SHARED((NPAD, D), jnp.float32),
        pltpu.SemaphoreType.DMA,
    ],
)


def _lin_body(feat_ref, w_ref, degt_ref, y_ref):
    x = jnp.dot(feat_ref[...], w_ref[...], preferred_element_type=jnp.float32)
    d = degt_ref[...]
    dis = lax.rsqrt(d[:, 0:1] + d[:, 1:2] + 1.0)
    y_ref[...] = x * dis


_lin_call = pl.pallas_call(
    _lin_body,
    grid=(GRID,),
    in_specs=[
        pl.BlockSpec((RB, D), lambda i: (i, 0)),
        pl.BlockSpec((D, D), lambda i: (0, 0)),
        pl.BlockSpec((RB, NC), lambda i: (i, 0)),
    ],
    out_specs=pl.BlockSpec((RB, D), lambda i: (i, 0)),
    out_shape=jax.ShapeDtypeStruct((N, D), jnp.float32),
)


def _fin_body(accp_ref, y_ref, degt_ref, b_ref, out_ref):
    d = degt_ref[...]
    dis = lax.rsqrt(d[:, 0:1] + d[:, 1:2] + 1.0)
    s = accp_ref[0] + accp_ref[1] + y_ref[...]
    out_ref[...] = jnp.tanh(dis * s + b_ref[...][None, :])


_fin_call = pl.pallas_call(
    _fin_body,
    grid=(GRID,),
    in_specs=[
        pl.BlockSpec((NC, RB, D), lambda i: (0, i, 0)),
        pl.BlockSpec((RB, D), lambda i: (i, 0)),
        pl.BlockSpec((RB, NC), lambda i: (i, 0)),
        pl.BlockSpec((D,), lambda i: (0,)),
    ],
    out_specs=pl.BlockSpec((RB, D), lambda i: (i, 0)),
    out_shape=jax.ShapeDtypeStruct((N, D), jnp.float32),
)


def kernel(feat, edge_index, edge_weight, W, b):
    pad_dst = N + (jnp.arange(EPAD, dtype=jnp.int32) % (NPAD - N))
    src = jnp.concatenate(
        [edge_index[0].astype(jnp.int32), jnp.zeros((EPAD,), jnp.int32)])
    dst = jnp.concatenate([edge_index[1].astype(jnp.int32), pad_dst])
    ew = jnp.concatenate(
        [edge_weight.astype(jnp.float32), jnp.zeros((EPAD,), jnp.float32)])
    dst3 = dst.reshape(NW, NB, B)
    degp = _deg_call(dst3, ew)
    degt = jnp.transpose(degp)
    y = _lin_call(feat, W, degt)
    accp = _agg_call(src, dst3, ew, y)
    return _fin_call(accp, y, degt, b)


# speedup vs baseline: 3.4094x; 3.4094x over previous
import functools

import jax
import jax.numpy as jnp
from jax import lax
from jax.experimental import pallas as pl
from jax.experimental.pallas import tpu as pltpu
from jax.experimental.pallas import tpu_sc as plsc

N = 10000
E = 320000
D = 128
NC = 2
NS = 16
L = 16
NW = NC * NS
B = 80
NB = 125
EPW = NB * B
CH = 25
NCH = NB // CH
CHE = CH * B
NPAD = 10240
ZCH = NPAD // NS
RB = 1000
GRID = N // RB

_mesh = plsc.VectorSubcoreMesh(
    core_axis_name="c", subcore_axis_name="s", num_cores=NC, num_subcores=NS)


def _zero16():
    return jnp.zeros((L,), jnp.float32)


def _deg_body(dst_hbm, ew_hbm, degp_hbm, dst_v, ew_v, zbuf, deg_sh):
    cid = lax.axis_index("c")
    sid = lax.axis_index("s")
    wid = sid * NC + cid
    pltpu.sync_copy(dst_hbm.at[wid], dst_v)
    pltpu.sync_copy(ew_hbm.at[pl.ds(wid * EPW, EPW)], ew_v)

    def zb(t, carry):
        zbuf[pl.ds(t * L, L)] = _zero16()
        return carry
    lax.fori_loop(0, ZCH // L, zb, 0)
    pltpu.sync_copy(zbuf, deg_sh.at[pl.ds(sid * ZCH, ZCH)])
    plsc.subcore_barrier()

    def body(j, carry):
        pltpu.sync_copy(ew_v.at[pl.ds(j * B, B)], deg_sh.at[dst_v.at[j]],
                        add=True)
        return carry
    lax.fori_loop(0, NB, body, 0)
    plsc.subcore_barrier()
    pltpu.sync_copy(deg_sh.at[pl.ds(sid * ZCH, ZCH)],
                    degp_hbm.at[cid, pl.ds(sid * ZCH, ZCH)])


_deg_call = pl.kernel(
    _deg_body,
    out_type=jax.ShapeDtypeStruct((NC, NPAD), jnp.float32),
    mesh=_mesh,
    scratch_types=[
        pltpu.VMEM((NB, B), jnp.int32),
        pltpu.VMEM((EPW,), jnp.float32),
        pltpu.VMEM((ZCH,), jnp.float32),
        pltpu.VMEM_SHARED((NPAD,), jnp.float32),
    ],
)


def _agg_body(src_hbm, dst_hbm, ew_hbm, y_hbm, accp_hbm,
              src_v, dst_v, ew_v, gb0, gb1, acc_sh, gs0, gs1):
    cid = lax.axis_index("c")
    sid = lax.axis_index("s")
    wid = sid * NC + cid
    gbufs = (gb0, gb1)
    gsems = (gs0, gs1)

    def _gather(j, p):
        return pltpu.make_async_copy(
            y_hbm.at[src_v.at[pl.ds(j * B, B)]], gbufs[p], gsems[p])

    def _scale_scatter(j, p):
        def scale(g, c2):
            wvec = ew_v[pl.ds(j * B + g * L, L)]
            for i in range(L):
                wv = lax.broadcast_in_dim(wvec[i], (L,), ())
                e = g * L + i
                for k in range(D // L):
                    sl = pl.ds(k * L, L)
                    gbufs[p][e, sl] = gbufs[p][e, sl] * wv
            return c2
        lax.fori_loop(0, B // L, scale, 0)
        pltpu.sync_copy(gbufs[p], acc_sh.at[dst_v.at[j]], add=True)

    def zb(t, carry):
        for k in range(D // L):
            gb0[t, pl.ds(k * L, L)] = _zero16()
        return carry
    lax.fori_loop(0, B, zb, 0)
    for z in range(ZCH // B):
        pltpu.sync_copy(gb0, acc_sh.at[pl.ds(sid * ZCH + z * B, B)])
    plsc.subcore_barrier()

    def chunk(co, carry0):
        pltpu.sync_copy(src_hbm.at[pl.ds(wid * EPW + co * CHE, CHE)], src_v)
        pltpu.sync_copy(dst_hbm.at[wid, co], dst_v)
        pltpu.sync_copy(ew_hbm.at[pl.ds(wid * EPW + co * CHE, CHE)], ew_v)
        _gather(0, 0).start()

        def pair(jo, carry):
            j0 = 2 * jo
            _gather(j0 + 1, 1).start()
            _gather(j0, 0).wait()
            _scale_scatter(j0, 0)
            _gather(j0 + 2, 0).start()
            _gather(j0 + 1, 1).wait()
            _scale_scatter(j0 + 1, 1)
            return carry
        lax.fori_loop(0, CH // 2, pair, 0)
        _gather(CH - 1, 0).wait()
        _scale_scatter(CH - 1, 0)
        return carry0
    lax.fori_loop(0, NCH, chunk, 0)
    plsc.subcore_barrier()
    pltpu.sync_copy(acc_sh.at[pl.ds(sid * ZCH, ZCH)],
                    accp_hbm.at[cid, pl.ds(sid * ZCH, ZCH)])


_agg_call = pl.kernel(
    _agg_body,
    out_type=jax.ShapeDtypeStruct((NC, NPAD, D), jnp.float32),
    mesh=_mesh,
    scratch_types=[
        pltpu.VMEM((CHE,), jnp.int32),
        pltpu.VMEM((CH, B), jnp.int32),
        pltpu.VMEM((CHE,), jnp.float32),
        pltpu.VMEM((B, D), jnp.float32),
        pltpu.VMEM((B, D), jnp.float32),
        pltpu.VMEM_SHARED((NPAD, D), jnp.float32),
        pltpu.SemaphoreType.DMA,
        pltpu.SemaphoreType.DMA,
    ],
)


def _lin_body(feat_ref, w_ref, degt_ref, y_ref):
    x = jnp.dot(feat_ref[...], w_ref[...], preferred_element_type=jnp.float32)
    d = degt_ref[...]
    dis = lax.rsqrt(d[:, 0:1] + d[:, 1:2] + 1.0)
    y_ref[...] = x * dis


_lin_call = pl.pallas_call(
    _lin_body,
    grid=(GRID,),
    in_specs=[
        pl.BlockSpec((RB, D), lambda i: (i, 0)),
        pl.BlockSpec((D, D), lambda i: (0, 0)),
        pl.BlockSpec((RB, NC), lambda i: (i, 0)),
    ],
    out_specs=pl.BlockSpec((RB, D), lambda i: (i, 0)),
    out_shape=jax.ShapeDtypeStruct((N, D), jnp.float32),
)


def _fin_body(accp_ref, y_ref, degt_ref, b_ref, out_ref):
    d = degt_ref[...]
    dis = lax.rsqrt(d[:, 0:1] + d[:, 1:2] + 1.0)
    s = accp_ref[0] + accp_ref[1] + y_ref[...]
    out_ref[...] = jnp.tanh(dis * s + b_ref[...][None, :])


_fin_call = pl.pallas_call(
    _fin_body,
    grid=(GRID,),
    in_specs=[
        pl.BlockSpec((NC, RB, D), lambda i: (0, i, 0)),
        pl.BlockSpec((RB, D), lambda i: (i, 0)),
        pl.BlockSpec((RB, NC), lambda i: (i, 0)),
        pl.BlockSpec((D,), lambda i: (0,)),
    ],
    out_specs=pl.BlockSpec((RB, D), lambda i: (i, 0)),
    out_shape=jax.ShapeDtypeStruct((N, D), jnp.float32),
)


def kernel(feat, edge_index, edge_weight, W, b):
    src = edge_index[0].astype(jnp.int32)
    dst = edge_index[1].astype(jnp.int32)
    ew = edge_weight.astype(jnp.float32)
    dst3 = dst.reshape(NW, NB, B)
    dst4 = dst.reshape(NW, NCH, CH, B)
    degp = _deg_call(dst3, ew)
    degt = jnp.transpose(degp)
    y = _lin_call(feat, W, degt)
    accp = _agg_call(src, dst4, ew, y)
    return _fin_call(accp, y, degt, b)


# speedup vs baseline: 3.6801x; 1.0794x over previous
import functools

import jax
import jax.numpy as jnp
from jax import lax
from jax.experimental import pallas as pl
from jax.experimental.pallas import tpu as pltpu
from jax.experimental.pallas import tpu_sc as plsc

N = 10000
E = 320000
D = 128
NC = 2
NS = 16
L = 16
NW = NC * NS
B = 80
NB = 125
EPW = NB * B
CH = 25
NCH = NB // CH
CHE = CH * B
NPAD = 10240
ZCH = NPAD // NS
RB = 1000
GRID = N // RB

_mesh = plsc.VectorSubcoreMesh(
    core_axis_name="c", subcore_axis_name="s", num_cores=NC, num_subcores=NS)


def _zero16():
    return jnp.zeros((L,), jnp.float32)


def _deg_body(dst_hbm, ew_hbm, degp_hbm, dst_v, ew_v, zbuf, deg_sh):
    cid = lax.axis_index("c")
    sid = lax.axis_index("s")
    wid = sid * NC + cid
    pltpu.sync_copy(dst_hbm.at[wid], dst_v)
    pltpu.sync_copy(ew_hbm.at[pl.ds(wid * EPW, EPW)], ew_v)

    def zb(t, carry):
        zbuf[pl.ds(t * L, L)] = _zero16()
        return carry
    lax.fori_loop(0, ZCH // L, zb, 0)
    pltpu.sync_copy(zbuf, deg_sh.at[pl.ds(sid * ZCH, ZCH)])
    plsc.subcore_barrier()

    def body(j, carry):
        pltpu.sync_copy(ew_v.at[pl.ds(j * B, B)], deg_sh.at[dst_v.at[j]],
                        add=True)
        return carry
    lax.fori_loop(0, NB, body, 0)
    plsc.subcore_barrier()
    pltpu.sync_copy(deg_sh.at[pl.ds(sid * ZCH, ZCH)],
                    degp_hbm.at[cid, pl.ds(sid * ZCH, ZCH)])


_deg_call = pl.kernel(
    _deg_body,
    out_type=jax.ShapeDtypeStruct((NC, NPAD), jnp.float32),
    mesh=_mesh,
    scratch_types=[
        pltpu.VMEM((NB, B), jnp.int32),
        pltpu.VMEM((EPW,), jnp.float32),
        pltpu.VMEM((ZCH,), jnp.float32),
        pltpu.VMEM_SHARED((NPAD,), jnp.float32),
    ],
)


def _agg_body(src_hbm, dst_hbm, ew_hbm, y_hbm, accp_hbm,
              src_v, dst_v, ew_v, gb0, gb1, gb2, acc_sh,
              gs0, gs1, gs2, ss0, ss1, ss2):
    cid = lax.axis_index("c")
    sid = lax.axis_index("s")
    wid = sid * NC + cid
    bufs = (gb0, gb1, gb2)
    gsems = (gs0, gs1, gs2)
    ssems = (ss0, ss1, ss2)

    def _gather(j, p):
        return pltpu.make_async_copy(
            y_hbm.at[src_v.at[pl.ds(j * B, B)]], bufs[p], gsems[p])

    def _scat_start(j, p):
        pltpu.async_copy(bufs[p], acc_sh.at[dst_v.at[j]], ssems[p],
                         add=True)

    def _scat_wait(j, p):
        pltpu.make_async_copy(bufs[p], acc_sh.at[dst_v.at[j]],
                              ssems[p]).wait()

    def _scale(j, p):
        def body(g, c2):
            wvec = ew_v[pl.ds(j * B + g * L, L)]
            for i in range(L):
                wv = lax.broadcast_in_dim(wvec[i], (L,), ())
                e = g * L + i
                for k in range(D // L):
                    sl = pl.ds(k * L, L)
                    bufs[p][e, sl] = bufs[p][e, sl] * wv
            return c2
        lax.fori_loop(0, B // L, body, 0)

    def zb(t, carry):
        for k in range(D // L):
            gb0[t, pl.ds(k * L, L)] = _zero16()
        return carry
    lax.fori_loop(0, B, zb, 0)
    for z in range(ZCH // B):
        pltpu.sync_copy(gb0, acc_sh.at[pl.ds(sid * ZCH + z * B, B)])
    plsc.subcore_barrier()

    def chunk(co, carry0):
        pltpu.sync_copy(src_hbm.at[pl.ds(wid * EPW + co * CHE, CHE)], src_v)
        pltpu.sync_copy(dst_hbm.at[wid, co], dst_v)
        pltpu.sync_copy(ew_hbm.at[pl.ds(wid * EPW + co * CHE, CHE)], ew_v)
        _gather(0, 0).start()
        _gather(1, 1).start()
        _gather(2, 2).start()
        _gather(0, 0).wait()
        _scale(0, 0)
        _scat_start(0, 0)
        _gather(1, 1).wait()
        _scale(1, 1)
        _scat_start(1, 1)
        _scat_wait(0, 0)
        _gather(3, 0).start()
        _gather(2, 2).wait()
        _scale(2, 2)
        _scat_start(2, 2)
        _scat_wait(1, 1)
        _gather(4, 1).start()

        def triple(jo, carry):
            for u in range(3):
                j = 3 * jo + u
                _gather(j, u).wait()
                _scale(j, u)
                _scat_start(j, u)
                _scat_wait(j - 1, (u + 2) % 3)

                @pl.when(j + 2 < CH)
                def _():
                    _gather(j + 2, (u + 2) % 3).start()
            return carry
        lax.fori_loop(1, CH // 3, triple, 0)
        _gather(CH - 1, (CH - 1) % 3).wait()
        _scale(CH - 1, (CH - 1) % 3)
        _scat_start(CH - 1, (CH - 1) % 3)
        _scat_wait(CH - 2, (CH - 2) % 3)
        _scat_wait(CH - 1, (CH - 1) % 3)
        return carry0
    lax.fori_loop(0, NCH, chunk, 0)
    plsc.subcore_barrier()
    pltpu.sync_copy(acc_sh.at[pl.ds(sid * ZCH, ZCH)],
                    accp_hbm.at[cid, pl.ds(sid * ZCH, ZCH)])


_agg_call = pl.kernel(
    _agg_body,
    out_type=jax.ShapeDtypeStruct((NC, NPAD, D), jnp.float32),
    mesh=_mesh,
    scratch_types=[
        pltpu.VMEM((CHE,), jnp.int32),
        pltpu.VMEM((CH, B), jnp.int32),
        pltpu.VMEM((CHE,), jnp.float32),
        pltpu.VMEM((B, D), jnp.float32),
        pltpu.VMEM((B, D), jnp.float32),
        pltpu.VMEM((B, D), jnp.float32),
        pltpu.VMEM_SHARED((NPAD, D), jnp.float32),
        pltpu.SemaphoreType.DMA,
        pltpu.SemaphoreType.DMA,
        pltpu.SemaphoreType.DMA,
        pltpu.SemaphoreType.DMA,
        pltpu.SemaphoreType.DMA,
        pltpu.SemaphoreType.DMA,
    ],
)


def _lin_body(feat_ref, w_ref, degt_ref, y_ref):
    x = jnp.dot(feat_ref[...], w_ref[...], preferred_element_type=jnp.float32)
    d = degt_ref[...]
    dis = lax.rsqrt(d[:, 0:1] + d[:, 1:2] + 1.0)
    y_ref[...] = x * dis


_lin_call = pl.pallas_call(
    _lin_body,
    grid=(GRID,),
    in_specs=[
        pl.BlockSpec((RB, D), lambda i: (i, 0)),
        pl.BlockSpec((D, D), lambda i: (0, 0)),
        pl.BlockSpec((RB, NC), lambda i: (i, 0)),
    ],
    out_specs=pl.BlockSpec((RB, D), lambda i: (i, 0)),
    out_shape=jax.ShapeDtypeStruct((N, D), jnp.float32),
)


def _fin_body(accp_ref, y_ref, degt_ref, b_ref, out_ref):
    d = degt_ref[...]
    dis = lax.rsqrt(d[:, 0:1] + d[:, 1:2] + 1.0)
    s = accp_ref[0] + accp_ref[1] + y_ref[...]
    out_ref[...] = jnp.tanh(dis * s + b_ref[...][None, :])


_fin_call = pl.pallas_call(
    _fin_body,
    grid=(GRID,),
    in_specs=[
        pl.BlockSpec((NC, RB, D), lambda i: (0, i, 0)),
        pl.BlockSpec((RB, D), lambda i: (i, 0)),
        pl.BlockSpec((RB, NC), lambda i: (i, 0)),
        pl.BlockSpec((D,), lambda i: (0,)),
    ],
    out_specs=pl.BlockSpec((RB, D), lambda i: (i, 0)),
    out_shape=jax.ShapeDtypeStruct((N, D), jnp.float32),
)


def kernel(feat, edge_index, edge_weight, W, b):
    src = edge_index[0].astype(jnp.int32)
    dst = edge_index[1].astype(jnp.int32)
    ew = edge_weight.astype(jnp.float32)
    dst3 = dst.reshape(NW, NB, B)
    dst4 = dst.reshape(NW, NCH, CH, B)
    degp = _deg_call(dst3, ew)
    degt = jnp.transpose(degp)
    y = _lin_call(feat, W, degt)
    accp = _agg_call(src, dst4, ew, y)
    return _fin_call(accp, y, degt, b)


# speedup vs baseline: 3.7724x; 1.0251x over previous
import functools

import jax
import jax.numpy as jnp
from jax import lax
from jax.experimental import pallas as pl
from jax.experimental.pallas import tpu as pltpu
from jax.experimental.pallas import tpu_sc as plsc

N = 10000
E = 320000
D = 128
NC = 2
NS = 16
L = 16
NW = NC * NS
B = 80
NB = 125
EPW = NB * B
CH = 25
NCH = NB // CH
CHE = CH * B
NPAD = 10240
ZCH = NPAD // NS
RB = 1000
GRID = N // RB

_mesh = plsc.VectorSubcoreMesh(
    core_axis_name="c", subcore_axis_name="s", num_cores=NC, num_subcores=NS)


def _zero16():
    return jnp.zeros((L,), jnp.float32)


def _deg_body(dst_hbm, ew_hbm, degp_hbm, dst_v, ew_v, zbuf, deg_sh):
    cid = lax.axis_index("c")
    sid = lax.axis_index("s")
    wid = sid * NC + cid
    pltpu.sync_copy(dst_hbm.at[wid], dst_v)
    pltpu.sync_copy(ew_hbm.at[pl.ds(wid * EPW, EPW)], ew_v)

    def zb(t, carry):
        zbuf[pl.ds(t * L, L)] = _zero16()
        return carry
    lax.fori_loop(0, ZCH // L, zb, 0)
    pltpu.sync_copy(zbuf, deg_sh.at[pl.ds(sid * ZCH, ZCH)])
    plsc.subcore_barrier()

    def body(j, carry):
        pltpu.sync_copy(ew_v.at[pl.ds(j * B, B)], deg_sh.at[dst_v.at[j]],
                        add=True)
        return carry
    lax.fori_loop(0, NB, body, 0)
    plsc.subcore_barrier()
    pltpu.sync_copy(deg_sh.at[pl.ds(sid * ZCH, ZCH)],
                    degp_hbm.at[cid, pl.ds(sid * ZCH, ZCH)])


_deg_call = pl.kernel(
    _deg_body,
    out_type=jax.ShapeDtypeStruct((NC, NPAD), jnp.float32),
    mesh=_mesh,
    scratch_types=[
        pltpu.VMEM((NB, B), jnp.int32),
        pltpu.VMEM((EPW,), jnp.float32),
        pltpu.VMEM((ZCH,), jnp.float32),
        pltpu.VMEM_SHARED((NPAD,), jnp.float32),
    ],
)


def _agg_body(src_hbm, dst_hbm, ew_hbm, y_hbm, accp_hbm,
              src_v, dst_v, ew_v, gb0, gb1, gb2, acc_sh,
              gs0, gs1, gs2, ss0, ss1, ss2, csem):
    cid = lax.axis_index("c")
    sid = lax.axis_index("s")
    wid = sid * NC + cid
    bufs = (gb0, gb1, gb2)
    gsems = (gs0, gs1, gs2)
    ssems = (ss0, ss1, ss2)

    def _gather(j, p):
        return pltpu.make_async_copy(
            y_hbm.at[src_v.at[pl.ds(j * B, B)]], bufs[p], gsems[p])

    def _scat_start(j, p):
        pltpu.async_copy(bufs[p], acc_sh.at[dst_v.at[j]], ssems[p],
                         add=True)

    def _scat_wait(j, p):
        pltpu.make_async_copy(bufs[p], acc_sh.at[dst_v.at[j]],
                              ssems[p]).wait()

    def _scale(j, p):
        def body(g, c2):
            wvec = ew_v[pl.ds(j * B + g * L, L)]
            for i in range(L):
                wv = lax.broadcast_in_dim(wvec[i], (L,), ())
                e = g * L + i
                for k in range(D // L):
                    sl = pl.ds(k * L, L)
                    bufs[p][e, sl] = bufs[p][e, sl] * wv
            return c2
        lax.fori_loop(0, B // L, body, 0)

    def zb(t, carry):
        for k in range(D // L):
            gb0[t, pl.ds(k * L, L)] = _zero16()
        return carry
    lax.fori_loop(0, B, zb, 0)
    for z in range(ZCH // B):
        pltpu.sync_copy(gb0, acc_sh.at[pl.ds(sid * ZCH + z * B, B)])
    plsc.subcore_barrier()

    def chunk(co, carry0):
        ca = pltpu.make_async_copy(
            src_hbm.at[pl.ds(wid * EPW + co * CHE, CHE)], src_v, csem)
        cb = pltpu.make_async_copy(dst_hbm.at[wid, co], dst_v, csem)
        cc = pltpu.make_async_copy(
            ew_hbm.at[pl.ds(wid * EPW + co * CHE, CHE)], ew_v, csem)
        ca.start()
        cb.start()
        cc.start()
        ca.wait()
        cb.wait()
        cc.wait()
        _gather(0, 0).start()
        _gather(1, 1).start()
        _gather(2, 2).start()
        _gather(0, 0).wait()
        _scale(0, 0)
        _scat_start(0, 0)
        _gather(1, 1).wait()
        _scale(1, 1)
        _scat_start(1, 1)
        _scat_wait(0, 0)
        _gather(3, 0).start()
        _gather(2, 2).wait()
        _scale(2, 2)
        _scat_start(2, 2)
        _scat_wait(1, 1)
        _gather(4, 1).start()

        def triple(jo, carry):
            for u in range(3):
                j = 3 * jo + u
                _gather(j, u).wait()
                _scale(j, u)
                _scat_start(j, u)
                _scat_wait(j - 1, (u + 2) % 3)

                @pl.when(j + 2 < CH)
                def _():
                    _gather(j + 2, (u + 2) % 3).start()
            return carry
        lax.fori_loop(1, CH // 3, triple, 0)
        _gather(CH - 1, (CH - 1) % 3).wait()
        _scale(CH - 1, (CH - 1) % 3)
        _scat_start(CH - 1, (CH - 1) % 3)
        _scat_wait(CH - 2, (CH - 2) % 3)
        _scat_wait(CH - 1, (CH - 1) % 3)
        return carry0
    lax.fori_loop(0, NCH, chunk, 0)
    plsc.subcore_barrier()
    pltpu.sync_copy(acc_sh.at[pl.ds(sid * ZCH, ZCH)],
                    accp_hbm.at[cid, pl.ds(sid * ZCH, ZCH)])


_agg_call = pl.kernel(
    _agg_body,
    out_type=jax.ShapeDtypeStruct((NC, NPAD, D), jnp.float32),
    mesh=_mesh,
    scratch_types=[
        pltpu.VMEM((CHE,), jnp.int32),
        pltpu.VMEM((CH, B), jnp.int32),
        pltpu.VMEM((CHE,), jnp.float32),
        pltpu.VMEM((B, D), jnp.float32),
        pltpu.VMEM((B, D), jnp.float32),
        pltpu.VMEM((B, D), jnp.float32),
        pltpu.VMEM_SHARED((NPAD, D), jnp.float32),
        pltpu.SemaphoreType.DMA,
        pltpu.SemaphoreType.DMA,
        pltpu.SemaphoreType.DMA,
        pltpu.SemaphoreType.DMA,
        pltpu.SemaphoreType.DMA,
        pltpu.SemaphoreType.DMA,
        pltpu.SemaphoreType.DMA,
    ],
)


def _lin_body(feat_ref, w_ref, degt_ref, y_ref):
    x = jnp.dot(feat_ref[...], w_ref[...], preferred_element_type=jnp.float32)
    d = degt_ref[...]
    dis = lax.rsqrt(d[:, 0:1] + d[:, 1:2] + 1.0)
    y_ref[...] = x * dis


_lin_call = pl.pallas_call(
    _lin_body,
    grid=(GRID,),
    in_specs=[
        pl.BlockSpec((RB, D), lambda i: (i, 0)),
        pl.BlockSpec((D, D), lambda i: (0, 0)),
        pl.BlockSpec((RB, NC), lambda i: (i, 0)),
    ],
    out_specs=pl.BlockSpec((RB, D), lambda i: (i, 0)),
    out_shape=jax.ShapeDtypeStruct((N, D), jnp.float32),
)


def _fin_body(accp_ref, y_ref, degt_ref, b_ref, out_ref):
    d = degt_ref[...]
    dis = lax.rsqrt(d[:, 0:1] + d[:, 1:2] + 1.0)
    s = accp_ref[0] + accp_ref[1] + y_ref[...]
    out_ref[...] = jnp.tanh(dis * s + b_ref[...][None, :])


_fin_call = pl.pallas_call(
    _fin_body,
    grid=(GRID,),
    in_specs=[
        pl.BlockSpec((NC, RB, D), lambda i: (0, i, 0)),
        pl.BlockSpec((RB, D), lambda i: (i, 0)),
        pl.BlockSpec((RB, NC), lambda i: (i, 0)),
        pl.BlockSpec((D,), lambda i: (0,)),
    ],
    out_specs=pl.BlockSpec((RB, D), lambda i: (i, 0)),
    out_shape=jax.ShapeDtypeStruct((N, D), jnp.float32),
)


def kernel(feat, edge_index, edge_weight, W, b):
    src = edge_index[0].astype(jnp.int32)
    dst = edge_index[1].astype(jnp.int32)
    ew = edge_weight.astype(jnp.float32)
    dst3 = dst.reshape(NW, NB, B)
    dst4 = dst.reshape(NW, NCH, CH, B)
    degp = _deg_call(dst3, ew)
    degt = jnp.transpose(degp)
    y = _lin_call(feat, W, degt)
    accp = _agg_call(src, dst4, ew, y)
    return _fin_call(accp, y, degt, b)
